# Initial kernel scaffold; baseline (speedup 1.0000x reference)
import jax
import jax.numpy as jnp
from jax.experimental import pallas as pl


def kernel(x, iInd, jInd):
    raise NotImplementedError("write your pallas kernel here")


# speedup vs baseline: 172.3761x; 172.3761x over previous
import functools

import jax
import jax.numpy as jnp
from jax import lax
from jax.experimental import pallas as pl
from jax.experimental.pallas import tpu as pltpu
from jax.experimental.pallas import tpu_sc as plsc

N_NODES = 100000
N_EDGES = 1600000
NC = 2
NS = 16
LANES = 16
CHW = 16
ROWS = 100096
CHUNK = 128
EPT = -(-N_EDGES // (NS * CHUNK)) * CHUNK
EPAD = EPT * NS
CPT = EPT // CHUNK
RPT = ROWS // NS


def _sc_body(iind_hbm, jind_hbm, table_hbm, zeros_hbm, out_hbm,
             acc, ii, jj, gi, gj, xi, xj, g, gn, sem_i, sem_j):
    cid = lax.axis_index("c")
    sid = lax.axis_index("s")
    pltpu.sync_copy(zeros_hbm.at[pl.ds(sid * RPT, RPT)],
                    acc.at[pl.ds(sid * RPT, RPT)])
    plsc.subcore_barrier()

    coff = cid * ROWS
    tile_base = sid * EPT

    def chunk_body(c, _):
        base = tile_base + c * CHUNK
        pltpu.sync_copy(iind_hbm.at[pl.ds(base, CHUNK)], ii)
        pltpu.sync_copy(jind_hbm.at[pl.ds(base, CHUNK)], jj)
        for s in range(CHUNK // LANES):
            sl = pl.ds(s * LANES, LANES)
            gi[sl] = ii[sl] + coff
            gj[sl] = jj[sl] + coff
        di = pltpu.async_copy(table_hbm.at[gi], xi, sem_i)
        dj = pltpu.async_copy(table_hbm.at[gj], xj, sem_j)
        di.wait()
        dj.wait()
        for r in range(CHUNK):
            a = xi[r, :]
            b = xj[r, :]
            g[r, :] = a - b
            gn[r, :] = b - a
        pltpu.sync_copy(g, acc.at[ii], add=True)
        pltpu.sync_copy(gn, acc.at[jj], add=True)
        return ()

    lax.fori_loop(0, CPT, chunk_body, (), unroll=False)
    plsc.subcore_barrier()
    pltpu.sync_copy(acc.at[pl.ds(sid * RPT, RPT)],
                    out_hbm.at[pl.ds(coff + sid * RPT, RPT)])


@functools.partial(
    pl.kernel,
    out_type=jax.ShapeDtypeStruct((NC * ROWS, CHW), jnp.float32),
    mesh=plsc.VectorSubcoreMesh(core_axis_name="c", subcore_axis_name="s"),
    scratch_types=[
        pltpu.VMEM_SHARED((ROWS, CHW), jnp.float32),
        pltpu.VMEM((CHUNK,), jnp.int32),
        pltpu.VMEM((CHUNK,), jnp.int32),
        pltpu.VMEM((CHUNK,), jnp.int32),
        pltpu.VMEM((CHUNK,), jnp.int32),
        pltpu.VMEM((CHUNK, CHW), jnp.float32),
        pltpu.VMEM((CHUNK, CHW), jnp.float32),
        pltpu.VMEM((CHUNK, CHW), jnp.float32),
        pltpu.VMEM((CHUNK, CHW), jnp.float32),
        pltpu.SemaphoreType.DMA,
        pltpu.SemaphoreType.DMA,
    ],
    compiler_params=pltpu.CompilerParams(use_tc_tiling_on_sc=False),
)
def _laplacian_sc(iind_hbm, jind_hbm, table_hbm, zeros_hbm, out_hbm, *scratch):
    _sc_body(iind_hbm, jind_hbm, table_hbm, zeros_hbm, out_hbm, *scratch)


def kernel(x, iInd, jInd):
    xf = x.reshape(NC * 12, N_NODES)
    xf = jnp.pad(xf, ((0, 0), (0, ROWS - N_NODES)))
    xt = xf.reshape(NC, 12, ROWS).transpose(0, 2, 1)
    table = jnp.pad(xt, ((0, 0), (0, 0), (0, CHW - 12))).reshape(NC * ROWS, CHW)
    pad = jnp.full((EPAD - N_EDGES,), N_NODES, dtype=jnp.int32)
    ii = jnp.concatenate([iInd.astype(jnp.int32), pad])
    jj = jnp.concatenate([jInd.astype(jnp.int32), pad])
    zeros = jnp.zeros((ROWS, CHW), dtype=jnp.float32)
    out_flat = _laplacian_sc(ii, jj, table, zeros)
    out = out_flat.reshape(NC, ROWS, CHW)[:, :N_NODES, :12]
    return out.transpose(0, 2, 1).reshape(1, 8, 3, N_NODES)


# speedup vs baseline: 267.9228x; 1.5543x over previous
import functools

import jax
import jax.numpy as jnp
from jax import lax
from jax.experimental import pallas as pl
from jax.experimental.pallas import tpu as pltpu
from jax.experimental.pallas import tpu_sc as plsc

N_NODES = 100000
N_EDGES = 1600000
NC = 2
NS = 16
LANES = 16
CHW = 16
ROWS = 100096
CHUNK = 128
BLK = 8
EPT = -(-N_EDGES // (NS * CHUNK * BLK)) * CHUNK * BLK
EPAD = EPT * NS
CPT = EPT // CHUNK
NBLK = CPT // BLK
RPT = ROWS // NS


def _sc_body(iind_hbm, jind_hbm, table_hbm, zeros_hbm, out_hbm,
             acc, ii3, jj3, gi3, gj3, xi, xj, g, gn,
             sem_ii, sem_jj, sem_gi, sem_gj):
    cid = lax.axis_index("c")
    sid = lax.axis_index("s")
    pltpu.sync_copy(zeros_hbm.at[pl.ds(sid * RPT, RPT)],
                    acc.at[pl.ds(sid * RPT, RPT)])
    plsc.subcore_barrier()

    coff = cid * ROWS
    rbase = sid * CPT

    def idx_rows(blk):
        return pl.ds(rbase + blk * BLK, BLK)

    for b in range(2):
        pltpu.async_copy(iind_hbm.at[idx_rows(b)], ii3.at[b], sem_ii)
        pltpu.async_copy(jind_hbm.at[idx_rows(b)], jj3.at[b], sem_jj)

    def block_body(blk, _):
        p = lax.rem(blk, 2)
        pltpu.make_async_copy(iind_hbm.at[idx_rows(blk)], ii3.at[p], sem_ii).wait()
        pltpu.make_async_copy(jind_hbm.at[idx_rows(blk)], jj3.at[p], sem_jj).wait()

        def off_body(k, _):
            for s in range(CHUNK // LANES):
                sl = pl.ds(s * LANES, LANES)
                gi3[p, k, sl] = ii3[p, k, sl] + coff
                gj3[p, k, sl] = jj3[p, k, sl] + coff
            return ()
        lax.fori_loop(0, BLK, off_body, (), unroll=True)

        for k in range(2):
            pltpu.async_copy(table_hbm.at[gi3.at[p, k]], xi.at[k], sem_gi)
            pltpu.async_copy(table_hbm.at[gj3.at[p, k]], xj.at[k], sem_gj)

        for k in range(BLK):
            q = k % 2
            pltpu.make_async_copy(table_hbm.at[gi3.at[p, k]], xi.at[q], sem_gi).wait()
            pltpu.make_async_copy(table_hbm.at[gj3.at[p, k]], xj.at[q], sem_gj).wait()

            def diff_body(r, _):
                a = xi[q, r, :]
                b = xj[q, r, :]
                g[r, :] = a - b
                gn[r, :] = b - a
                return ()
            lax.fori_loop(0, CHUNK, diff_body, (), unroll=8)

            if k < BLK - 2:
                pltpu.async_copy(table_hbm.at[gi3.at[p, k + 2]], xi.at[q], sem_gi)
                pltpu.async_copy(table_hbm.at[gj3.at[p, k + 2]], xj.at[q], sem_gj)

            pltpu.sync_copy(g, acc.at[ii3.at[p, k]], add=True)
            pltpu.sync_copy(gn, acc.at[jj3.at[p, k]], add=True)

        nxt = lax.min(blk + 2, NBLK - 1)
        pltpu.async_copy(iind_hbm.at[idx_rows(nxt)], ii3.at[p], sem_ii)
        pltpu.async_copy(jind_hbm.at[idx_rows(nxt)], jj3.at[p], sem_jj)
        return ()

    lax.fori_loop(0, NBLK, block_body, (), unroll=False)

    for b in range(2):
        pltpu.make_async_copy(iind_hbm.at[idx_rows(0)], ii3.at[b], sem_ii).wait()
        pltpu.make_async_copy(jind_hbm.at[idx_rows(0)], jj3.at[b], sem_jj).wait()

    plsc.subcore_barrier()
    pltpu.sync_copy(acc.at[pl.ds(sid * RPT, RPT)],
                    out_hbm.at[pl.ds(coff + sid * RPT, RPT)])


@functools.partial(
    pl.kernel,
    out_type=jax.ShapeDtypeStruct((NC * ROWS, CHW), jnp.float32),
    mesh=plsc.VectorSubcoreMesh(core_axis_name="c", subcore_axis_name="s"),
    scratch_types=[
        pltpu.VMEM_SHARED((ROWS, CHW), jnp.float32),
        pltpu.VMEM((2, BLK, CHUNK), jnp.int32),
        pltpu.VMEM((2, BLK, CHUNK), jnp.int32),
        pltpu.VMEM((2, BLK, CHUNK), jnp.int32),
        pltpu.VMEM((2, BLK, CHUNK), jnp.int32),
        pltpu.VMEM((2, CHUNK, CHW), jnp.float32),
        pltpu.VMEM((2, CHUNK, CHW), jnp.float32),
        pltpu.VMEM((CHUNK, CHW), jnp.float32),
        pltpu.VMEM((CHUNK, CHW), jnp.float32),
        pltpu.SemaphoreType.DMA,
        pltpu.SemaphoreType.DMA,
        pltpu.SemaphoreType.DMA,
        pltpu.SemaphoreType.DMA,
    ],
    compiler_params=pltpu.CompilerParams(use_tc_tiling_on_sc=False),
)
def _laplacian_sc(iind_hbm, jind_hbm, table_hbm, zeros_hbm, out_hbm, *scratch):
    _sc_body(iind_hbm, jind_hbm, table_hbm, zeros_hbm, out_hbm, *scratch)


def kernel(x, iInd, jInd):
    xf = x.reshape(NC * 12, N_NODES)
    xf = jnp.pad(xf, ((0, 0), (0, ROWS - N_NODES)))
    xt = xf.reshape(NC, 12, ROWS).transpose(0, 2, 1)
    table = jnp.pad(xt, ((0, 0), (0, 0), (0, CHW - 12))).reshape(NC * ROWS, CHW)
    pad = jnp.full((EPAD - N_EDGES,), N_NODES, dtype=jnp.int32)
    ii = jnp.concatenate([iInd.astype(jnp.int32), pad]).reshape(EPAD // CHUNK, CHUNK)
    jj = jnp.concatenate([jInd.astype(jnp.int32), pad]).reshape(EPAD // CHUNK, CHUNK)
    zeros = jnp.zeros((ROWS, CHW), dtype=jnp.float32)
    out_flat = _laplacian_sc(ii, jj, table, zeros)
    out = out_flat.reshape(NC, ROWS, CHW)[:, :N_NODES, :12]
    return out.transpose(0, 2, 1).reshape(1, 8, 3, N_NODES)


# speedup vs baseline: 298.8182x; 1.1153x over previous
import functools

import jax
import jax.numpy as jnp
from jax import lax
from jax.experimental import pallas as pl
from jax.experimental.pallas import tpu as pltpu
from jax.experimental.pallas import tpu_sc as plsc

N_NODES = 100000
N_EDGES = 1600000
NC = 2
NS = 16
LANES = 16
CHW = 16
ROWS = 100096
CHUNK = 128
BLK = 8
EPT = -(-N_EDGES // (NS * CHUNK * BLK)) * CHUNK * BLK
EPAD = EPT * NS
CPT = EPT // CHUNK
NBLK = CPT // BLK
RPT = ROWS // NS


def _sc_body(iind_hbm, jind_hbm, table_hbm, zeros_hbm, out_hbm,
             acc, ii3, jj3, gi3, gj3, xi, xj, g, gn,
             sem_ii, sem_jj, sem_gi, sem_gj, sem_si, sem_sj):
    cid = lax.axis_index("c")
    sid = lax.axis_index("s")
    pltpu.sync_copy(zeros_hbm.at[pl.ds(sid * RPT, RPT)],
                    acc.at[pl.ds(sid * RPT, RPT)])
    plsc.subcore_barrier()

    coff = cid * ROWS
    rbase = sid * CPT

    def idx_rows(blk):
        return pl.ds(rbase + blk * BLK, BLK)

    for b in range(2):
        pltpu.async_copy(iind_hbm.at[idx_rows(b)], ii3.at[b], sem_ii)
        pltpu.async_copy(jind_hbm.at[idx_rows(b)], jj3.at[b], sem_jj)

    def block_body(blk, _):
        p = lax.rem(blk, 2)
        pltpu.make_async_copy(iind_hbm.at[idx_rows(blk)], ii3.at[p], sem_ii).wait()
        pltpu.make_async_copy(jind_hbm.at[idx_rows(blk)], jj3.at[p], sem_jj).wait()

        def off_body(k, _):
            for s in range(CHUNK // LANES):
                sl = pl.ds(s * LANES, LANES)
                gi3[p, k, sl] = ii3[p, k, sl] + coff
                gj3[p, k, sl] = jj3[p, k, sl] + coff
            return ()
        lax.fori_loop(0, BLK, off_body, (), unroll=True)

        for k in range(2):
            pltpu.async_copy(table_hbm.at[gi3.at[p, k]], xi.at[k], sem_gi)
            pltpu.async_copy(table_hbm.at[gj3.at[p, k]], xj.at[k], sem_gj)

        for k in range(BLK):
            q = k % 2
            pltpu.make_async_copy(table_hbm.at[gi3.at[p, k]], xi.at[q], sem_gi).wait()
            pltpu.make_async_copy(table_hbm.at[gj3.at[p, k]], xj.at[q], sem_gj).wait()

            def drain_scatter():
                pltpu.make_async_copy(g.at[q], acc.at[ii3.at[p, k]], sem_si).wait()
                pltpu.make_async_copy(gn.at[q], acc.at[jj3.at[p, k]], sem_sj).wait()
            if k < 2:
                @pl.when(blk > 0)
                def _():
                    drain_scatter()
            else:
                drain_scatter()

            def diff_body(r, _):
                a = xi[q, r, :]
                b = xj[q, r, :]
                g[q, r, :] = a - b
                gn[q, r, :] = b - a
                return ()
            lax.fori_loop(0, CHUNK, diff_body, (), unroll=8)

            if k < BLK - 2:
                pltpu.async_copy(table_hbm.at[gi3.at[p, k + 2]], xi.at[q], sem_gi)
                pltpu.async_copy(table_hbm.at[gj3.at[p, k + 2]], xj.at[q], sem_gj)

            pltpu.async_copy(g.at[q], acc.at[ii3.at[p, k]], sem_si, add=True)
            pltpu.async_copy(gn.at[q], acc.at[jj3.at[p, k]], sem_sj, add=True)

        nxt = lax.min(blk + 2, NBLK - 1)
        pltpu.async_copy(iind_hbm.at[idx_rows(nxt)], ii3.at[p], sem_ii)
        pltpu.async_copy(jind_hbm.at[idx_rows(nxt)], jj3.at[p], sem_jj)
        return ()

    lax.fori_loop(0, NBLK, block_body, (), unroll=False)

    for q in range(2):
        pltpu.make_async_copy(g.at[q], acc.at[ii3.at[0, 0]], sem_si).wait()
        pltpu.make_async_copy(gn.at[q], acc.at[jj3.at[0, 0]], sem_sj).wait()
    for b in range(2):
        pltpu.make_async_copy(iind_hbm.at[idx_rows(0)], ii3.at[b], sem_ii).wait()
        pltpu.make_async_copy(jind_hbm.at[idx_rows(0)], jj3.at[b], sem_jj).wait()

    plsc.subcore_barrier()
    pltpu.sync_copy(acc.at[pl.ds(sid * RPT, RPT)],
                    out_hbm.at[pl.ds(coff + sid * RPT, RPT)])


@functools.partial(
    pl.kernel,
    out_type=jax.ShapeDtypeStruct((NC * ROWS, CHW), jnp.float32),
    mesh=plsc.VectorSubcoreMesh(core_axis_name="c", subcore_axis_name="s"),
    scratch_types=[
        pltpu.VMEM_SHARED((ROWS, CHW), jnp.float32),
        pltpu.VMEM((2, BLK, CHUNK), jnp.int32),
        pltpu.VMEM((2, BLK, CHUNK), jnp.int32),
        pltpu.VMEM((2, BLK, CHUNK), jnp.int32),
        pltpu.VMEM((2, BLK, CHUNK), jnp.int32),
        pltpu.VMEM((2, CHUNK, CHW), jnp.float32),
        pltpu.VMEM((2, CHUNK, CHW), jnp.float32),
        pltpu.VMEM((2, CHUNK, CHW), jnp.float32),
        pltpu.VMEM((2, CHUNK, CHW), jnp.float32),
        pltpu.SemaphoreType.DMA,
        pltpu.SemaphoreType.DMA,
        pltpu.SemaphoreType.DMA,
        pltpu.SemaphoreType.DMA,
        pltpu.SemaphoreType.DMA,
        pltpu.SemaphoreType.DMA,
    ],
    compiler_params=pltpu.CompilerParams(use_tc_tiling_on_sc=False),
)
def _laplacian_sc(iind_hbm, jind_hbm, table_hbm, zeros_hbm, out_hbm, *scratch):
    _sc_body(iind_hbm, jind_hbm, table_hbm, zeros_hbm, out_hbm, *scratch)


def kernel(x, iInd, jInd):
    xf = x.reshape(NC * 12, N_NODES)
    xf = jnp.pad(xf, ((0, 0), (0, ROWS - N_NODES)))
    xt = xf.reshape(NC, 12, ROWS).transpose(0, 2, 1)
    table = jnp.pad(xt, ((0, 0), (0, 0), (0, CHW - 12))).reshape(NC * ROWS, CHW)
    pad = jnp.full((EPAD - N_EDGES,), N_NODES, dtype=jnp.int32)
    ii = jnp.concatenate([iInd.astype(jnp.int32), pad]).reshape(EPAD // CHUNK, CHUNK)
    jj = jnp.concatenate([jInd.astype(jnp.int32), pad]).reshape(EPAD // CHUNK, CHUNK)
    zeros = jnp.zeros((ROWS, CHW), dtype=jnp.float32)
    out_flat = _laplacian_sc(ii, jj, table, zeros)
    out = out_flat.reshape(NC, ROWS, CHW)[:, :N_NODES, :12]
    return out.transpose(0, 2, 1).reshape(1, 8, 3, N_NODES)


# speedup vs baseline: 326.5527x; 1.0928x over previous
import functools

import jax
import jax.numpy as jnp
from jax import lax
from jax.experimental import pallas as pl
from jax.experimental.pallas import tpu as pltpu
from jax.experimental.pallas import tpu_sc as plsc

N_NODES = 100000
N_EDGES = 1600000
NC = 2
NS = 16
LANES = 16
CHH = 12
CHW = 16
ROWS = 100096
CHUNK = 128
BLK = 8
EPT = -(-N_EDGES // (NS * CHUNK * BLK)) * CHUNK * BLK
EPAD = EPT * NS
CPT = EPT // CHUNK
NBLK = CPT // BLK
RPT = ROWS // NS
ECH = 128
NECH = RPT // ECH
ETAIL = RPT - NECH * ECH
LTAIL = (N_NODES - (NS - 1) * RPT) - NECH * ECH


def _sc_body(iind_hbm, jind_hbm, table_hbm, zeros_hbm, out_hbm,
             acc, ii3, jj3, gi3, gj3, xi, xj, g, gn, cbuf, tbuf,
             sem_ii, sem_jj, sem_gi, sem_gj, sem_si, sem_sj, sem_ep):
    cid = lax.axis_index("c")
    sid = lax.axis_index("s")
    pltpu.sync_copy(zeros_hbm.at[pl.ds(sid * RPT, RPT)],
                    acc.at[pl.ds(sid * RPT, RPT)])
    plsc.subcore_barrier()

    coff = cid * ROWS
    rbase = sid * CPT

    def idx_rows(blk):
        return pl.ds(rbase + blk * BLK, BLK)

    for b in range(2):
        pltpu.async_copy(iind_hbm.at[idx_rows(b)], ii3.at[b], sem_ii)
        pltpu.async_copy(jind_hbm.at[idx_rows(b)], jj3.at[b], sem_jj)

    def block_body(blk, _):
        p = lax.rem(blk, 2)
        pltpu.make_async_copy(iind_hbm.at[idx_rows(blk)], ii3.at[p], sem_ii).wait()
        pltpu.make_async_copy(jind_hbm.at[idx_rows(blk)], jj3.at[p], sem_jj).wait()

        def off_body(k, _):
            for s in range(CHUNK // LANES):
                sl = pl.ds(s * LANES, LANES)
                gi3[p, k, sl] = ii3[p, k, sl] + coff
                gj3[p, k, sl] = jj3[p, k, sl] + coff
            return ()
        lax.fori_loop(0, BLK, off_body, (), unroll=True)

        for k in range(2):
            pltpu.async_copy(table_hbm.at[gi3.at[p, k]], xi.at[k], sem_gi)
            pltpu.async_copy(table_hbm.at[gj3.at[p, k]], xj.at[k], sem_gj)

        for k in range(BLK):
            q = k % 2
            pltpu.make_async_copy(table_hbm.at[gi3.at[p, k]], xi.at[q], sem_gi).wait()
            pltpu.make_async_copy(table_hbm.at[gj3.at[p, k]], xj.at[q], sem_gj).wait()

            def drain_scatter():
                pltpu.make_async_copy(g.at[q], acc.at[ii3.at[p, k]], sem_si).wait()
                pltpu.make_async_copy(gn.at[q], acc.at[jj3.at[p, k]], sem_sj).wait()
            if k < 2:
                @pl.when(blk > 0)
                def _():
                    drain_scatter()
            else:
                drain_scatter()

            def diff_body(r, _):
                a = xi[q, r, :]
                b = xj[q, r, :]
                g[q, r, :] = a - b
                gn[q, r, :] = b - a
                return ()
            lax.fori_loop(0, CHUNK, diff_body, (), unroll=8)

            if k < BLK - 2:
                pltpu.async_copy(table_hbm.at[gi3.at[p, k + 2]], xi.at[q], sem_gi)
                pltpu.async_copy(table_hbm.at[gj3.at[p, k + 2]], xj.at[q], sem_gj)

            pltpu.async_copy(g.at[q], acc.at[ii3.at[p, k]], sem_si, add=True)
            pltpu.async_copy(gn.at[q], acc.at[jj3.at[p, k]], sem_sj, add=True)

        nxt = lax.min(blk + 2, NBLK - 1)
        pltpu.async_copy(iind_hbm.at[idx_rows(nxt)], ii3.at[p], sem_ii)
        pltpu.async_copy(jind_hbm.at[idx_rows(nxt)], jj3.at[p], sem_jj)
        return ()

    lax.fori_loop(0, NBLK, block_body, (), unroll=False)

    for q in range(2):
        pltpu.make_async_copy(g.at[q], acc.at[ii3.at[0, 0]], sem_si).wait()
        pltpu.make_async_copy(gn.at[q], acc.at[jj3.at[0, 0]], sem_sj).wait()
    for b in range(2):
        pltpu.make_async_copy(iind_hbm.at[idx_rows(0)], ii3.at[b], sem_ii).wait()
        pltpu.make_async_copy(jind_hbm.at[idx_rows(0)], jj3.at[b], sem_jj).wait()

    plsc.subcore_barrier()

    iota = lax.iota(jnp.int32, LANES)
    zero16 = iota * 0
    mask12 = iota < CHH
    chb = cid * CHH
    nb0 = sid * RPT

    def pull_and_transpose(c, pe):
        pltpu.sync_copy(acc.at[pl.ds(nb0 + c * ECH, ECH)], cbuf)

        def t_body(r, _):
            vals = cbuf[r, :]
            plsc.store_scatter(tbuf.at[pe], [iota, zero16 + r], vals,
                               mask=mask12)
            return ()
        lax.fori_loop(0, ECH, t_body, (), unroll=8)

    def full_out(c, pe):
        return pltpu.make_async_copy(
            tbuf.at[pe],
            out_hbm.at[pl.ds(chb, CHH), pl.ds(nb0 + c * ECH, ECH)], sem_ep)

    def ep_body(c, _):
        pe = lax.rem(c, 2)
        @pl.when(c >= 2)
        def _():
            full_out(c - 2, pe).wait()
        pull_and_transpose(c, pe)
        full_out(c, pe).start()
        return ()

    lax.fori_loop(0, NECH, ep_body, (), unroll=False)
    for c in range(NECH - 2, NECH):
        full_out(c, c % 2).wait()

    pull_and_transpose(NECH, 0)

    @pl.when(sid < NS - 1)
    def _():
        pltpu.sync_copy(tbuf.at[0, :, pl.ds(0, ETAIL)],
                        out_hbm.at[pl.ds(chb, CHH),
                                   pl.ds(nb0 + NECH * ECH, ETAIL)])

    @pl.when(sid == NS - 1)
    def _():
        pltpu.sync_copy(tbuf.at[0, :, pl.ds(0, LTAIL)],
                        out_hbm.at[pl.ds(chb, CHH),
                                   pl.ds(nb0 + NECH * ECH, LTAIL)])


@functools.partial(
    pl.kernel,
    out_type=jax.ShapeDtypeStruct((NC * CHH, N_NODES), jnp.float32),
    mesh=plsc.VectorSubcoreMesh(core_axis_name="c", subcore_axis_name="s"),
    scratch_types=[
        pltpu.VMEM_SHARED((ROWS + ECH, CHW), jnp.float32),
        pltpu.VMEM((2, BLK, CHUNK), jnp.int32),
        pltpu.VMEM((2, BLK, CHUNK), jnp.int32),
        pltpu.VMEM((2, BLK, CHUNK), jnp.int32),
        pltpu.VMEM((2, BLK, CHUNK), jnp.int32),
        pltpu.VMEM((2, CHUNK, CHW), jnp.float32),
        pltpu.VMEM((2, CHUNK, CHW), jnp.float32),
        pltpu.VMEM((2, CHUNK, CHW), jnp.float32),
        pltpu.VMEM((2, CHUNK, CHW), jnp.float32),
        pltpu.VMEM((ECH, CHW), jnp.float32),
        pltpu.VMEM((2, CHH, ECH), jnp.float32),
        pltpu.SemaphoreType.DMA,
        pltpu.SemaphoreType.DMA,
        pltpu.SemaphoreType.DMA,
        pltpu.SemaphoreType.DMA,
        pltpu.SemaphoreType.DMA,
        pltpu.SemaphoreType.DMA,
        pltpu.SemaphoreType.DMA,
    ],
    compiler_params=pltpu.CompilerParams(use_tc_tiling_on_sc=False,
                                         needs_layout_passes=False),
)
def _laplacian_sc(iind_hbm, jind_hbm, table_hbm, zeros_hbm, out_hbm, *scratch):
    _sc_body(iind_hbm, jind_hbm, table_hbm, zeros_hbm, out_hbm, *scratch)


def kernel(x, iInd, jInd):
    xf = x.reshape(NC * CHH, N_NODES)
    xf = jnp.pad(xf, ((0, 0), (0, ROWS - N_NODES)))
    xt = xf.reshape(NC, CHH, ROWS).transpose(0, 2, 1)
    table = jnp.pad(xt, ((0, 0), (0, 0), (0, CHW - CHH))).reshape(NC * ROWS, CHW)
    pad = jnp.full((EPAD - N_EDGES,), N_NODES, dtype=jnp.int32)
    ii = jnp.concatenate([iInd.astype(jnp.int32), pad]).reshape(EPAD // CHUNK, CHUNK)
    jj = jnp.concatenate([jInd.astype(jnp.int32), pad]).reshape(EPAD // CHUNK, CHUNK)
    zeros = jnp.zeros((ROWS, CHW), dtype=jnp.float32)
    out24 = _laplacian_sc(ii, jj, table, zeros)
    return out24.reshape(1, 8, 3, N_NODES)


# speedup vs baseline: 360.3823x; 1.1036x over previous
import functools

import jax
import jax.numpy as jnp
from jax import lax
from jax.experimental import pallas as pl
from jax.experimental.pallas import tpu as pltpu
from jax.experimental.pallas import tpu_sc as plsc

N_NODES = 100000
N_EDGES = 1600000
NC = 2
NS = 16
LANES = 16
CHH = 12
CHW = 16
ROWS = 100096
CHUNK = 128
BLK = 8
EPT = -(-N_EDGES // (NS * CHUNK * BLK)) * CHUNK * BLK
EPAD = EPT * NS
CPT = EPT // CHUNK
NBLK = CPT // BLK
RPT = ROWS // NS
ECH = 128
NECH = RPT // ECH
ETAIL = RPT - NECH * ECH
LTAIL = (N_NODES - (NS - 1) * RPT) - NECH * ECH


def _sc_body(iind_hbm, jind_hbm, x_hbm, zeros_hbm, out_hbm, table_hbm,
             acc, ii3, jj3, gi3, gj3, xi, xj, g, gn, cbuf, tbuf,
             sem_ii, sem_jj, sem_gi, sem_gj, sem_si, sem_sj, sem_ep):
    cid = lax.axis_index("c")
    sid = lax.axis_index("s")
    iota = lax.iota(jnp.int32, LANES)
    zero16 = iota * 0
    mask12 = iota < CHH
    chb = cid * CHH
    nb0 = sid * RPT
    coff = cid * ROWS

    def build_chunk(c, width):
        pltpu.sync_copy(x_hbm.at[pl.ds(chb, CHH), pl.ds(nb0 + c * ECH, width)],
                        tbuf.at[0, :, pl.ds(0, width)])

        def b_body(s, _):
            rows16 = iota + s * LANES
            for ch in range(CHH):
                vals = tbuf[0, ch, pl.ds(s * LANES, LANES)]
                plsc.store_scatter(xi.at[0], [rows16, zero16 + ch], vals)
            return ()
        lax.fori_loop(0, width // LANES, b_body, (), unroll=False)
        pltpu.sync_copy(xi.at[0, pl.ds(0, width)],
                        table_hbm.at[pl.ds(coff + nb0 + c * ECH, width)])

    def p0_body(c, _):
        build_chunk(c, ECH)
        return ()
    lax.fori_loop(0, NECH, p0_body, (), unroll=False)

    @pl.when(sid < NS - 1)
    def _():
        build_chunk(NECH, ETAIL)

    @pl.when(sid == NS - 1)
    def _():
        build_chunk(NECH, LTAIL)

    pltpu.sync_copy(zeros_hbm.at[pl.ds(sid * RPT, RPT)],
                    acc.at[pl.ds(sid * RPT, RPT)])
    plsc.subcore_barrier()

    rbase = sid * CPT

    def idx_rows(blk):
        return pl.ds(rbase + blk * BLK, BLK)

    for b in range(2):
        pltpu.async_copy(iind_hbm.at[idx_rows(b)], ii3.at[b], sem_ii)
        pltpu.async_copy(jind_hbm.at[idx_rows(b)], jj3.at[b], sem_jj)

    def block_body(blk, _):
        p = lax.rem(blk, 2)
        pltpu.make_async_copy(iind_hbm.at[idx_rows(blk)], ii3.at[p], sem_ii).wait()
        pltpu.make_async_copy(jind_hbm.at[idx_rows(blk)], jj3.at[p], sem_jj).wait()

        def off_body(k, _):
            for s in range(CHUNK // LANES):
                sl = pl.ds(s * LANES, LANES)
                gi3[p, k, sl] = ii3[p, k, sl] + coff
                gj3[p, k, sl] = jj3[p, k, sl] + coff
            return ()
        lax.fori_loop(0, BLK, off_body, (), unroll=True)

        for k in range(2):
            pltpu.async_copy(table_hbm.at[gi3.at[p, k]], xi.at[k], sem_gi)
            pltpu.async_copy(table_hbm.at[gj3.at[p, k]], xj.at[k], sem_gj)

        for k in range(BLK):
            q = k % 2
            pltpu.make_async_copy(table_hbm.at[gi3.at[p, k]], xi.at[q], sem_gi).wait()
            pltpu.make_async_copy(table_hbm.at[gj3.at[p, k]], xj.at[q], sem_gj).wait()

            def drain_scatter():
                pltpu.make_async_copy(g.at[q], acc.at[ii3.at[p, k]], sem_si).wait()
                pltpu.make_async_copy(gn.at[q], acc.at[jj3.at[p, k]], sem_sj).wait()
            if k < 2:
                @pl.when(blk > 0)
                def _():
                    drain_scatter()
            else:
                drain_scatter()

            def diff_body(r, _):
                a = xi[q, r, :]
                b = xj[q, r, :]
                g[q, r, :] = a - b
                gn[q, r, :] = b - a
                return ()
            lax.fori_loop(0, CHUNK, diff_body, (), unroll=8)

            if k < BLK - 2:
                pltpu.async_copy(table_hbm.at[gi3.at[p, k + 2]], xi.at[q], sem_gi)
                pltpu.async_copy(table_hbm.at[gj3.at[p, k + 2]], xj.at[q], sem_gj)

            pltpu.async_copy(g.at[q], acc.at[ii3.at[p, k]], sem_si, add=True)
            pltpu.async_copy(gn.at[q], acc.at[jj3.at[p, k]], sem_sj, add=True)

        nxt = lax.min(blk + 2, NBLK - 1)
        pltpu.async_copy(iind_hbm.at[idx_rows(nxt)], ii3.at[p], sem_ii)
        pltpu.async_copy(jind_hbm.at[idx_rows(nxt)], jj3.at[p], sem_jj)
        return ()

    lax.fori_loop(0, NBLK, block_body, (), unroll=False)

    for q in range(2):
        pltpu.make_async_copy(g.at[q], acc.at[ii3.at[0, 0]], sem_si).wait()
        pltpu.make_async_copy(gn.at[q], acc.at[jj3.at[0, 0]], sem_sj).wait()
    for b in range(2):
        pltpu.make_async_copy(iind_hbm.at[idx_rows(0)], ii3.at[b], sem_ii).wait()
        pltpu.make_async_copy(jind_hbm.at[idx_rows(0)], jj3.at[b], sem_jj).wait()

    plsc.subcore_barrier()

    def pull_and_transpose(c, pe):
        pltpu.sync_copy(acc.at[pl.ds(nb0 + c * ECH, ECH)], cbuf)

        def t_body(r, _):
            vals = cbuf[r, :]
            plsc.store_scatter(tbuf.at[pe], [iota, zero16 + r], vals,
                               mask=mask12)
            return ()
        lax.fori_loop(0, ECH, t_body, (), unroll=8)

    def full_out(c, pe):
        return pltpu.make_async_copy(
            tbuf.at[pe],
            out_hbm.at[pl.ds(chb, CHH), pl.ds(nb0 + c * ECH, ECH)], sem_ep)

    def ep_body(c, _):
        pe = lax.rem(c, 2)
        @pl.when(c >= 2)
        def _():
            full_out(c - 2, pe).wait()
        pull_and_transpose(c, pe)
        full_out(c, pe).start()
        return ()

    lax.fori_loop(0, NECH, ep_body, (), unroll=False)
    for c in range(NECH - 2, NECH):
        full_out(c, c % 2).wait()

    pull_and_transpose(NECH, 0)

    @pl.when(sid < NS - 1)
    def _():
        pltpu.sync_copy(tbuf.at[0, :, pl.ds(0, ETAIL)],
                        out_hbm.at[pl.ds(chb, CHH),
                                   pl.ds(nb0 + NECH * ECH, ETAIL)])

    @pl.when(sid == NS - 1)
    def _():
        pltpu.sync_copy(tbuf.at[0, :, pl.ds(0, LTAIL)],
                        out_hbm.at[pl.ds(chb, CHH),
                                   pl.ds(nb0 + NECH * ECH, LTAIL)])


@functools.partial(
    pl.kernel,
    out_type=(jax.ShapeDtypeStruct((NC * CHH, N_NODES), jnp.float32),
              jax.ShapeDtypeStruct((NC * ROWS, CHW), jnp.float32)),
    mesh=plsc.VectorSubcoreMesh(core_axis_name="c", subcore_axis_name="s"),
    scratch_types=[
        pltpu.VMEM_SHARED((ROWS + ECH, CHW), jnp.float32),
        pltpu.VMEM((2, BLK, CHUNK), jnp.int32),
        pltpu.VMEM((2, BLK, CHUNK), jnp.int32),
        pltpu.VMEM((2, BLK, CHUNK), jnp.int32),
        pltpu.VMEM((2, BLK, CHUNK), jnp.int32),
        pltpu.VMEM((2, CHUNK, CHW), jnp.float32),
        pltpu.VMEM((2, CHUNK, CHW), jnp.float32),
        pltpu.VMEM((2, CHUNK, CHW), jnp.float32),
        pltpu.VMEM((2, CHUNK, CHW), jnp.float32),
        pltpu.VMEM((ECH, CHW), jnp.float32),
        pltpu.VMEM((2, CHH, ECH), jnp.float32),
        pltpu.SemaphoreType.DMA,
        pltpu.SemaphoreType.DMA,
        pltpu.SemaphoreType.DMA,
        pltpu.SemaphoreType.DMA,
        pltpu.SemaphoreType.DMA,
        pltpu.SemaphoreType.DMA,
        pltpu.SemaphoreType.DMA,
    ],
    compiler_params=pltpu.CompilerParams(use_tc_tiling_on_sc=False,
                                         needs_layout_passes=False),
)
def _laplacian_sc(iind_hbm, jind_hbm, x_hbm, zeros_hbm, out_hbm, table_hbm,
                  *scratch):
    _sc_body(iind_hbm, jind_hbm, x_hbm, zeros_hbm, out_hbm, table_hbm, *scratch)


def kernel(x, iInd, jInd):
    x24 = x.reshape(NC * CHH, N_NODES)
    pad = jnp.full((EPAD - N_EDGES,), N_NODES, dtype=jnp.int32)
    ii = jnp.concatenate([iInd.astype(jnp.int32), pad]).reshape(EPAD // CHUNK, CHUNK)
    jj = jnp.concatenate([jInd.astype(jnp.int32), pad]).reshape(EPAD // CHUNK, CHUNK)
    zeros = jnp.zeros((ROWS, CHW), dtype=jnp.float32)
    out24, _ = _laplacian_sc(ii, jj, x24, zeros)
    return out24.reshape(1, 8, 3, N_NODES)


# speedup vs baseline: 362.6173x; 1.0062x over previous
import functools

import jax
import jax.numpy as jnp
from jax import lax
from jax.experimental import pallas as pl
from jax.experimental.pallas import tpu as pltpu
from jax.experimental.pallas import tpu_sc as plsc

N_NODES = 100000
N_EDGES = 1600000
NC = 2
NS = 16
LANES = 16
CHH = 12
CHW = 16
ROWS = 100096
CHUNK = 128
BLK = 8
IROWS = N_EDGES // CHUNK
RPW = IROWS // NS
REXT = IROWS - RPW * NS
NBLK = RPW // BLK
RPT = ROWS // NS
ECH = 128
NECH = RPT // ECH
ETAIL = RPT - NECH * ECH
LTAIL = (N_NODES - (NS - 1) * RPT) - NECH * ECH


def _sc_body(iind_hbm, jind_hbm, x_hbm, zeros_hbm, out_hbm, table_hbm,
             acc, ii3, jj3, gi3, gj3, xi, xj, g, gn, cbuf, tbuf,
             sem_ii, sem_jj, sem_gi, sem_gj, sem_si, sem_sj, sem_ep):
    cid = lax.axis_index("c")
    sid = lax.axis_index("s")
    iota = lax.iota(jnp.int32, LANES)
    zero16 = iota * 0
    mask12 = iota < CHH
    chb = cid * CHH
    nb0 = sid * RPT
    coff = cid * ROWS

    def build_chunk(c, width):
        pltpu.sync_copy(x_hbm.at[pl.ds(chb, CHH), pl.ds(nb0 + c * ECH, width)],
                        tbuf.at[0, :, pl.ds(0, width)])

        def b_body(s, _):
            rows16 = iota + s * LANES
            for ch in range(CHH):
                vals = tbuf[0, ch, pl.ds(s * LANES, LANES)]
                plsc.store_scatter(xi.at[0], [rows16, zero16 + ch], vals)
            return ()
        lax.fori_loop(0, width // LANES, b_body, (), unroll=False)
        pltpu.sync_copy(xi.at[0, pl.ds(0, width)],
                        table_hbm.at[pl.ds(coff + nb0 + c * ECH, width)])

    def p0_body(c, _):
        build_chunk(c, ECH)
        return ()
    lax.fori_loop(0, NECH, p0_body, (), unroll=False)

    @pl.when(sid < NS - 1)
    def _():
        build_chunk(NECH, ETAIL)

    @pl.when(sid == NS - 1)
    def _():
        build_chunk(NECH, LTAIL)

    pltpu.sync_copy(zeros_hbm.at[pl.ds(sid * RPT, RPT)],
                    acc.at[pl.ds(sid * RPT, RPT)])
    plsc.subcore_barrier()

    rbase = sid * RPW + lax.min(sid, REXT)
    ntail = jnp.where(sid < REXT, RPW + 1, RPW) - NBLK * BLK

    def idx_rows(blk):
        return pl.ds(rbase + blk * BLK, BLK)

    for b in range(2):
        pltpu.async_copy(iind_hbm.at[idx_rows(b)], ii3.at[b], sem_ii)
        pltpu.async_copy(jind_hbm.at[idx_rows(b)], jj3.at[b], sem_jj)

    def block_body(blk, _):
        p = lax.rem(blk, 2)
        pltpu.make_async_copy(iind_hbm.at[idx_rows(blk)], ii3.at[p], sem_ii).wait()
        pltpu.make_async_copy(jind_hbm.at[idx_rows(blk)], jj3.at[p], sem_jj).wait()

        def off_body(k, _):
            for s in range(CHUNK // LANES):
                sl = pl.ds(s * LANES, LANES)
                gi3[p, k, sl] = ii3[p, k, sl] + coff
                gj3[p, k, sl] = jj3[p, k, sl] + coff
            return ()
        lax.fori_loop(0, BLK, off_body, (), unroll=True)

        for k in range(2):
            pltpu.async_copy(table_hbm.at[gi3.at[p, k]], xi.at[k], sem_gi)
            pltpu.async_copy(table_hbm.at[gj3.at[p, k]], xj.at[k], sem_gj)

        for k in range(BLK):
            q = k % 2
            pltpu.make_async_copy(table_hbm.at[gi3.at[p, k]], xi.at[q], sem_gi).wait()
            pltpu.make_async_copy(table_hbm.at[gj3.at[p, k]], xj.at[q], sem_gj).wait()

            def drain_scatter():
                pltpu.make_async_copy(g.at[q], acc.at[ii3.at[p, k]], sem_si).wait()
                pltpu.make_async_copy(gn.at[q], acc.at[jj3.at[p, k]], sem_sj).wait()
            if k < 2:
                @pl.when(blk > 0)
                def _():
                    drain_scatter()
            else:
                drain_scatter()

            def diff_body(r, _):
                a = xi[q, r, :]
                b = xj[q, r, :]
                g[q, r, :] = a - b
                gn[q, r, :] = b - a
                return ()
            lax.fori_loop(0, CHUNK, diff_body, (), unroll=8)

            if k < BLK - 2:
                pltpu.async_copy(table_hbm.at[gi3.at[p, k + 2]], xi.at[q], sem_gi)
                pltpu.async_copy(table_hbm.at[gj3.at[p, k + 2]], xj.at[q], sem_gj)

            pltpu.async_copy(g.at[q], acc.at[ii3.at[p, k]], sem_si, add=True)
            pltpu.async_copy(gn.at[q], acc.at[jj3.at[p, k]], sem_sj, add=True)

        nxt = lax.min(blk + 2, NBLK - 1)
        pltpu.async_copy(iind_hbm.at[idx_rows(nxt)], ii3.at[p], sem_ii)
        pltpu.async_copy(jind_hbm.at[idx_rows(nxt)], jj3.at[p], sem_jj)
        return ()

    lax.fori_loop(0, NBLK, block_body, (), unroll=False)

    for q in range(2):
        pltpu.make_async_copy(g.at[q], acc.at[ii3.at[0, 0]], sem_si).wait()
        pltpu.make_async_copy(gn.at[q], acc.at[jj3.at[0, 0]], sem_sj).wait()
    for b in range(2):
        pltpu.make_async_copy(iind_hbm.at[idx_rows(0)], ii3.at[b], sem_ii).wait()
        pltpu.make_async_copy(jind_hbm.at[idx_rows(0)], jj3.at[b], sem_jj).wait()

    tl0 = rbase + NBLK * BLK
    pltpu.sync_copy(iind_hbm.at[pl.ds(tl0, BLK - 3)], ii3.at[0, pl.ds(0, BLK - 3)])
    pltpu.sync_copy(jind_hbm.at[pl.ds(tl0, BLK - 3)], jj3.at[0, pl.ds(0, BLK - 3)])

    @pl.when(sid < REXT)
    def _():
        pltpu.sync_copy(iind_hbm.at[pl.ds(tl0 + BLK - 3, 1)],
                        ii3.at[0, pl.ds(BLK - 3, 1)])
        pltpu.sync_copy(jind_hbm.at[pl.ds(tl0 + BLK - 3, 1)],
                        jj3.at[0, pl.ds(BLK - 3, 1)])

    def tail_body(k, _):
        for s in range(CHUNK // LANES):
            sl = pl.ds(s * LANES, LANES)
            gi3[0, k, sl] = ii3[0, k, sl] + coff
            gj3[0, k, sl] = jj3[0, k, sl] + coff
        pltpu.async_copy(table_hbm.at[gi3.at[0, k]], xi.at[0], sem_gi).wait()
        pltpu.async_copy(table_hbm.at[gj3.at[0, k]], xj.at[0], sem_gj).wait()

        def diff_body(r, _):
            a = xi[0, r, :]
            b = xj[0, r, :]
            g[0, r, :] = a - b
            gn[0, r, :] = b - a
            return ()
        lax.fori_loop(0, CHUNK, diff_body, (), unroll=8)
        pltpu.sync_copy(g.at[0], acc.at[ii3.at[0, k]], add=True)
        pltpu.sync_copy(gn.at[0], acc.at[jj3.at[0, k]], add=True)
        return ()

    lax.fori_loop(0, ntail, tail_body, ())

    plsc.subcore_barrier()

    def pull_and_transpose(c, pe):
        pltpu.sync_copy(acc.at[pl.ds(nb0 + c * ECH, ECH)], cbuf)

        def t_body(r, _):
            vals = cbuf[r, :]
            plsc.store_scatter(tbuf.at[pe], [iota, zero16 + r], vals,
                               mask=mask12)
            return ()
        lax.fori_loop(0, ECH, t_body, (), unroll=8)

    def full_out(c, pe):
        return pltpu.make_async_copy(
            tbuf.at[pe],
            out_hbm.at[pl.ds(chb, CHH), pl.ds(nb0 + c * ECH, ECH)], sem_ep)

    def ep_body(c, _):
        pe = lax.rem(c, 2)
        @pl.when(c >= 2)
        def _():
            full_out(c - 2, pe).wait()
        pull_and_transpose(c, pe)
        full_out(c, pe).start()
        return ()

    lax.fori_loop(0, NECH, ep_body, (), unroll=False)
    for c in range(NECH - 2, NECH):
        full_out(c, c % 2).wait()

    pull_and_transpose(NECH, 0)

    @pl.when(sid < NS - 1)
    def _():
        pltpu.sync_copy(tbuf.at[0, :, pl.ds(0, ETAIL)],
                        out_hbm.at[pl.ds(chb, CHH),
                                   pl.ds(nb0 + NECH * ECH, ETAIL)])

    @pl.when(sid == NS - 1)
    def _():
        pltpu.sync_copy(tbuf.at[0, :, pl.ds(0, LTAIL)],
                        out_hbm.at[pl.ds(chb, CHH),
                                   pl.ds(nb0 + NECH * ECH, LTAIL)])


@functools.partial(
    pl.kernel,
    out_type=(jax.ShapeDtypeStruct((NC * CHH, N_NODES), jnp.float32),
              jax.ShapeDtypeStruct((NC * ROWS, CHW), jnp.float32)),
    mesh=plsc.VectorSubcoreMesh(core_axis_name="c", subcore_axis_name="s"),
    scratch_types=[
        pltpu.VMEM_SHARED((ROWS + ECH, CHW), jnp.float32),
        pltpu.VMEM((2, BLK, CHUNK), jnp.int32),
        pltpu.VMEM((2, BLK, CHUNK), jnp.int32),
        pltpu.VMEM((2, BLK, CHUNK), jnp.int32),
        pltpu.VMEM((2, BLK, CHUNK), jnp.int32),
        pltpu.VMEM((2, CHUNK, CHW), jnp.float32),
        pltpu.VMEM((2, CHUNK, CHW), jnp.float32),
        pltpu.VMEM((2, CHUNK, CHW), jnp.float32),
        pltpu.VMEM((2, CHUNK, CHW), jnp.float32),
        pltpu.VMEM((ECH, CHW), jnp.float32),
        pltpu.VMEM((2, CHH, ECH), jnp.float32),
        pltpu.SemaphoreType.DMA,
        pltpu.SemaphoreType.DMA,
        pltpu.SemaphoreType.DMA,
        pltpu.SemaphoreType.DMA,
        pltpu.SemaphoreType.DMA,
        pltpu.SemaphoreType.DMA,
        pltpu.SemaphoreType.DMA,
    ],
    compiler_params=pltpu.CompilerParams(use_tc_tiling_on_sc=False,
                                         needs_layout_passes=False),
)
def _laplacian_sc(iind_hbm, jind_hbm, x_hbm, zeros_hbm, out_hbm, table_hbm,
                  *scratch):
    _sc_body(iind_hbm, jind_hbm, x_hbm, zeros_hbm, out_hbm, table_hbm, *scratch)


def kernel(x, iInd, jInd):
    x24 = x.reshape(NC * CHH, N_NODES)
    ii = iInd.astype(jnp.int32).reshape(IROWS, CHUNK)
    jj = jInd.astype(jnp.int32).reshape(IROWS, CHUNK)
    zeros = jnp.zeros((ROWS, CHW), dtype=jnp.float32)
    out24, _ = _laplacian_sc(ii, jj, x24, zeros)
    return out24.reshape(1, 8, 3, N_NODES)
